# Initial kernel scaffold; baseline (speedup 1.0000x reference)
import functools

import jax
import jax.numpy as jnp
from jax import lax
from jax.experimental import pallas as pl
from jax.experimental.pallas import tpu as pltpu
from jax.experimental.pallas import tpu_sc as plsc

N = 10000
D = 128
E = 320000

NP = 10240
CHUNK = 128
NW = 32
CPW = 80
EP = NW * CPW * CHUNK
RS = NP // 16

_mesh = plsc.VectorSubcoreMesh(core_axis_name="c", subcore_axis_name="s")


@functools.partial(
    pl.kernel,
    out_type=jax.ShapeDtypeStruct((2, NP, 16), jnp.float32),
    mesh=_mesh,
    scratch_types=[
        pltpu.VMEM((CPW, CHUNK), jnp.int32),
        pltpu.VMEM((CHUNK, 16), jnp.float32),
        pltpu.VMEM_SHARED((NP, 16), jnp.float32),
    ],
)
def _hist_kernel(col_hbm, ones_hbm, zeros_hbm, out_hbm, col_v, ones_v, acc_sh):
    c = lax.axis_index("c")
    s = lax.axis_index("s")
    w = s * 2 + c
    pltpu.sync_copy(col_hbm.at[pl.ds(w * CPW, CPW)], col_v)
    pltpu.sync_copy(ones_hbm, ones_v)
    pltpu.sync_copy(zeros_hbm.at[pl.ds(s * RS, RS)], acc_sh.at[pl.ds(s * RS, RS)])
    plsc.subcore_barrier()

    @pl.loop(0, CPW)
    def _(j):
        pltpu.sync_copy(ones_v, acc_sh.at[col_v.at[j]], add=True)

    plsc.subcore_barrier()
    pltpu.sync_copy(acc_sh.at[pl.ds(s * RS, RS)], out_hbm.at[c, pl.ds(s * RS, RS)])


@functools.partial(
    pl.kernel,
    out_type=jax.ShapeDtypeStruct((2, NP, D), jnp.float32),
    mesh=_mesh,
    scratch_types=[
        pltpu.VMEM((CPW, CHUNK), jnp.int32),
        pltpu.VMEM((CPW, CHUNK), jnp.int32),
        pltpu.VMEM((CHUNK, D), jnp.float32),
        pltpu.VMEM((CHUNK, D), jnp.float32),
        pltpu.VMEM_SHARED((NP, D), jnp.float32),
        pltpu.SemaphoreType.DMA,
        pltpu.SemaphoreType.DMA,
    ],
)
def _agg_kernel(hs_hbm, row_hbm, col_hbm, zeros_hbm, out_hbm,
                row_v, col_v, buf0, buf1, acc_sh, sem0, sem1):
    c = lax.axis_index("c")
    s = lax.axis_index("s")
    w = s * 2 + c
    pltpu.sync_copy(row_hbm.at[pl.ds(w * CPW, CPW)], row_v)
    pltpu.sync_copy(col_hbm.at[pl.ds(w * CPW, CPW)], col_v)
    bufs = (buf0, buf1)
    sems = (sem0, sem1)
    for p in range(2):
        pltpu.make_async_copy(hs_hbm.at[row_v.at[p]], bufs[p], sems[p]).start()
    pltpu.sync_copy(zeros_hbm.at[pl.ds(s * RS, RS)], acc_sh.at[pl.ds(s * RS, RS)])
    plsc.subcore_barrier()

    @pl.loop(0, CPW, step=2)
    def _(j):
        for p in range(2):
            jj = j + p
            pltpu.make_async_copy(hs_hbm.at[row_v.at[jj]], bufs[p], sems[p]).wait()
            pltpu.sync_copy(bufs[p], acc_sh.at[col_v.at[jj]], add=True)

            @pl.when(jj + 2 < CPW)
            def _():
                pltpu.make_async_copy(
                    hs_hbm.at[row_v.at[jj + 2]], bufs[p], sems[p]).start()

    plsc.subcore_barrier()
    pltpu.sync_copy(acc_sh.at[pl.ds(s * RS, RS)], out_hbm.at[c, pl.ds(s * RS, RS)])


_BLK = 1024
_GRID = NP // _BLK


def _mm_scale_body(x_ref, w_ref, ha_ref, hb_ref, h_ref, hs_ref, dis_ref):
    h = jnp.dot(x_ref[...], w_ref[...], preferred_element_type=jnp.float32,
                precision=lax.Precision.HIGHEST)
    d16 = lax.rsqrt(ha_ref[...] + hb_ref[...] + 1.0)
    d = d16[:, 0:1]
    h_ref[...] = h
    hs_ref[...] = h * d
    dis_ref[...] = d16


_mm_scale = pl.pallas_call(
    _mm_scale_body,
    grid=(_GRID,),
    in_specs=[
        pl.BlockSpec((_BLK, D), lambda i: (i, 0)),
        pl.BlockSpec((D, D), lambda i: (0, 0)),
        pl.BlockSpec((_BLK, 16), lambda i: (i, 0)),
        pl.BlockSpec((_BLK, 16), lambda i: (i, 0)),
    ],
    out_specs=[
        pl.BlockSpec((_BLK, D), lambda i: (i, 0)),
        pl.BlockSpec((_BLK, D), lambda i: (i, 0)),
        pl.BlockSpec((_BLK, 16), lambda i: (i, 0)),
    ],
    out_shape=[
        jax.ShapeDtypeStruct((NP, D), jnp.float32),
        jax.ShapeDtypeStruct((NP, D), jnp.float32),
        jax.ShapeDtypeStruct((NP, 16), jnp.float32),
    ],
)


def _combine_mm_body(aa_ref, ab_ref, dis_ref, h1_ref, b_ref, w_ref,
                     h2_ref, hs2_ref):
    d = dis_ref[...][:, 0:1]
    z = d * (aa_ref[...] + ab_ref[...]) + (d * d) * h1_ref[...] + b_ref[...]
    r = jnp.maximum(z, 0.0)
    h2 = jnp.dot(r, w_ref[...], preferred_element_type=jnp.float32,
                 precision=lax.Precision.HIGHEST)
    h2_ref[...] = h2
    hs2_ref[...] = h2 * d


_combine_mm = pl.pallas_call(
    _combine_mm_body,
    grid=(_GRID,),
    in_specs=[
        pl.BlockSpec((_BLK, D), lambda i: (i, 0)),
        pl.BlockSpec((_BLK, D), lambda i: (i, 0)),
        pl.BlockSpec((_BLK, 16), lambda i: (i, 0)),
        pl.BlockSpec((_BLK, D), lambda i: (i, 0)),
        pl.BlockSpec((1, D), lambda i: (0, 0)),
        pl.BlockSpec((D, D), lambda i: (0, 0)),
    ],
    out_specs=[
        pl.BlockSpec((_BLK, D), lambda i: (i, 0)),
        pl.BlockSpec((_BLK, D), lambda i: (i, 0)),
    ],
    out_shape=[
        jax.ShapeDtypeStruct((NP, D), jnp.float32),
        jax.ShapeDtypeStruct((NP, D), jnp.float32),
    ],
)


def _final_body(aa_ref, ab_ref, dis_ref, h2_ref, b_ref, out_ref):
    d = dis_ref[...][:, 0:1]
    out_ref[...] = (d * (aa_ref[...] + ab_ref[...])
                    + (d * d) * h2_ref[...] + b_ref[...])


_final = pl.pallas_call(
    _final_body,
    grid=(_GRID,),
    in_specs=[
        pl.BlockSpec((_BLK, D), lambda i: (i, 0)),
        pl.BlockSpec((_BLK, D), lambda i: (i, 0)),
        pl.BlockSpec((_BLK, 16), lambda i: (i, 0)),
        pl.BlockSpec((_BLK, D), lambda i: (i, 0)),
        pl.BlockSpec((1, D), lambda i: (0, 0)),
    ],
    out_specs=pl.BlockSpec((_BLK, D), lambda i: (i, 0)),
    out_shape=jax.ShapeDtypeStruct((NP, D), jnp.float32),
)


def kernel(x, edge_index, W1, b1, W2, b2):
    row = edge_index[0]
    col = edge_index[1]
    pad = jnp.full((EP - E,), N, jnp.int32)
    row_p = jnp.concatenate([row, pad]).reshape(EP // CHUNK, CHUNK)
    col_p = jnp.concatenate([col, pad]).reshape(EP // CHUNK, CHUNK)
    x_p = jnp.pad(x, ((0, NP - N), (0, 0)))
    zeros128 = jnp.zeros((NP, D), jnp.float32)
    zeros16 = jnp.zeros((NP, 16), jnp.float32)
    ones16 = jnp.ones((CHUNK, 16), jnp.float32)
    b1r = b1.reshape(1, D)
    b2r = b2.reshape(1, D)

    hist = _hist_kernel(col_p, ones16, zeros16)
    h1, hs1, dis16 = _mm_scale(x_p, W1, hist[0], hist[1])
    acc1 = _agg_kernel(hs1, row_p, col_p, zeros128)
    h2, hs2 = _combine_mm(acc1[0], acc1[1], dis16, h1, b1r, W2)
    acc2 = _agg_kernel(hs2, row_p, col_p, zeros128)
    out = _final(acc2[0], acc2[1], dis16, h2, b2r)
    return out[:N]


# speedup vs baseline: 8.7742x; 8.7742x over previous
import functools

import jax
import jax.numpy as jnp
from jax import lax
from jax.experimental import pallas as pl
from jax.experimental.pallas import tpu as pltpu
from jax.experimental.pallas import tpu_sc as plsc

N = 10000
D = 128
E = 320000

NP = 10240
CHUNK = 128
NW = 32
CPW = 80
EP = NW * CPW * CHUNK
RS = NP // 16

_mesh = plsc.VectorSubcoreMesh(core_axis_name="c", subcore_axis_name="s")


@functools.partial(
    pl.kernel,
    out_type=jax.ShapeDtypeStruct((2, NP, D), jnp.float32),
    mesh=_mesh,
    scratch_types=[
        pltpu.VMEM((CPW, CHUNK), jnp.int32),
        pltpu.VMEM((CHUNK, D), jnp.float32),
        pltpu.VMEM_SHARED((NP, D), jnp.float32),
    ],
)
def _hist_kernel(col_hbm, ones_hbm, zeros_hbm, out_hbm, col_v, ones_v, acc_sh):
    c = lax.axis_index("c")
    s = lax.axis_index("s")
    w = s * 2 + c
    pltpu.sync_copy(col_hbm.at[pl.ds(w * CPW, CPW)], col_v)
    pltpu.sync_copy(ones_hbm, ones_v)
    pltpu.sync_copy(zeros_hbm.at[pl.ds(s * RS, RS)], acc_sh.at[pl.ds(s * RS, RS)])
    plsc.subcore_barrier()

    @pl.loop(0, CPW)
    def _(j):
        pltpu.sync_copy(ones_v, acc_sh.at[col_v.at[j]], add=True)

    plsc.subcore_barrier()
    pltpu.sync_copy(acc_sh.at[pl.ds(s * RS, RS)], out_hbm.at[c, pl.ds(s * RS, RS)])


CPH = CPW // 2


@functools.partial(
    pl.kernel,
    out_type=jax.ShapeDtypeStruct((2, NP, D), jnp.float32),
    mesh=_mesh,
    scratch_types=[
        pltpu.VMEM((CPH, CHUNK), jnp.int32),
        pltpu.VMEM((CPH, CHUNK), jnp.int32),
        pltpu.VMEM((CHUNK, D), jnp.float32),
        pltpu.VMEM((CHUNK, D), jnp.float32),
        pltpu.VMEM_SHARED((NP, D), jnp.float32),
        pltpu.SemaphoreType.DMA,
        pltpu.SemaphoreType.DMA,
    ],
)
def _agg_kernel(hs_hbm, row_hbm, col_hbm, zeros_hbm, out_hbm,
                row_v, col_v, buf0, buf1, acc_sh, sem0, sem1):
    c = lax.axis_index("c")
    s = lax.axis_index("s")
    w = s * 2 + c
    bufs = (buf0, buf1)
    sems = (sem0, sem1)
    pltpu.sync_copy(zeros_hbm.at[pl.ds(s * RS, RS)], acc_sh.at[pl.ds(s * RS, RS)])
    plsc.subcore_barrier()

    for half in range(2):
        base = w * CPW + half * CPH
        pltpu.sync_copy(row_hbm.at[pl.ds(base, CPH)], row_v)
        pltpu.sync_copy(col_hbm.at[pl.ds(base, CPH)], col_v)
        for p in range(2):
            pltpu.make_async_copy(hs_hbm.at[row_v.at[p]], bufs[p], sems[p]).start()

        @pl.loop(0, CPH, step=2)
        def _(j):
            for p in range(2):
                jj = j + p
                pltpu.make_async_copy(
                    hs_hbm.at[row_v.at[jj]], bufs[p], sems[p]).wait()
                pltpu.sync_copy(bufs[p], acc_sh.at[col_v.at[jj]], add=True)

                @pl.when(jj + 2 < CPH)
                def _():
                    pltpu.make_async_copy(
                        hs_hbm.at[row_v.at[jj + 2]], bufs[p], sems[p]).start()

    plsc.subcore_barrier()
    pltpu.sync_copy(acc_sh.at[pl.ds(s * RS, RS)], out_hbm.at[c, pl.ds(s * RS, RS)])


_BLK = 1024
_GRID = NP // _BLK


def _mm_scale_body(x_ref, w_ref, ha_ref, hb_ref, h_ref, hs_ref, dis_ref):
    h = jnp.dot(x_ref[...], w_ref[...], preferred_element_type=jnp.float32,
                precision=lax.Precision.HIGHEST)
    dfull = lax.rsqrt(ha_ref[...] + hb_ref[...] + 1.0)
    d = dfull[:, 0:1]
    h_ref[...] = h
    hs_ref[...] = h * d
    dis_ref[...] = dfull[:, :16]


_mm_scale = pl.pallas_call(
    _mm_scale_body,
    grid=(_GRID,),
    in_specs=[
        pl.BlockSpec((_BLK, D), lambda i: (i, 0)),
        pl.BlockSpec((D, D), lambda i: (0, 0)),
        pl.BlockSpec((_BLK, D), lambda i: (i, 0)),
        pl.BlockSpec((_BLK, D), lambda i: (i, 0)),
    ],
    out_specs=[
        pl.BlockSpec((_BLK, D), lambda i: (i, 0)),
        pl.BlockSpec((_BLK, D), lambda i: (i, 0)),
        pl.BlockSpec((_BLK, 16), lambda i: (i, 0)),
    ],
    out_shape=[
        jax.ShapeDtypeStruct((NP, D), jnp.float32),
        jax.ShapeDtypeStruct((NP, D), jnp.float32),
        jax.ShapeDtypeStruct((NP, 16), jnp.float32),
    ],
)


def _combine_mm_body(aa_ref, ab_ref, dis_ref, h1_ref, b_ref, w_ref,
                     h2_ref, hs2_ref):
    d = dis_ref[...][:, 0:1]
    z = d * (aa_ref[...] + ab_ref[...]) + (d * d) * h1_ref[...] + b_ref[...]
    r = jnp.maximum(z, 0.0)
    h2 = jnp.dot(r, w_ref[...], preferred_element_type=jnp.float32,
                 precision=lax.Precision.HIGHEST)
    h2_ref[...] = h2
    hs2_ref[...] = h2 * d


_combine_mm = pl.pallas_call(
    _combine_mm_body,
    grid=(_GRID,),
    in_specs=[
        pl.BlockSpec((_BLK, D), lambda i: (i, 0)),
        pl.BlockSpec((_BLK, D), lambda i: (i, 0)),
        pl.BlockSpec((_BLK, 16), lambda i: (i, 0)),
        pl.BlockSpec((_BLK, D), lambda i: (i, 0)),
        pl.BlockSpec((1, D), lambda i: (0, 0)),
        pl.BlockSpec((D, D), lambda i: (0, 0)),
    ],
    out_specs=[
        pl.BlockSpec((_BLK, D), lambda i: (i, 0)),
        pl.BlockSpec((_BLK, D), lambda i: (i, 0)),
    ],
    out_shape=[
        jax.ShapeDtypeStruct((NP, D), jnp.float32),
        jax.ShapeDtypeStruct((NP, D), jnp.float32),
    ],
)


def _final_body(aa_ref, ab_ref, dis_ref, h2_ref, b_ref, out_ref):
    d = dis_ref[...][:, 0:1]
    out_ref[...] = (d * (aa_ref[...] + ab_ref[...])
                    + (d * d) * h2_ref[...] + b_ref[...])


_final = pl.pallas_call(
    _final_body,
    grid=(_GRID,),
    in_specs=[
        pl.BlockSpec((_BLK, D), lambda i: (i, 0)),
        pl.BlockSpec((_BLK, D), lambda i: (i, 0)),
        pl.BlockSpec((_BLK, 16), lambda i: (i, 0)),
        pl.BlockSpec((_BLK, D), lambda i: (i, 0)),
        pl.BlockSpec((1, D), lambda i: (0, 0)),
    ],
    out_specs=pl.BlockSpec((_BLK, D), lambda i: (i, 0)),
    out_shape=jax.ShapeDtypeStruct((NP, D), jnp.float32),
)


def kernel(x, edge_index, W1, b1, W2, b2):
    row = edge_index[0]
    col = edge_index[1]
    pad = jnp.full((EP - E,), N, jnp.int32)
    row_p = jnp.concatenate([row, pad]).reshape(EP // CHUNK, CHUNK)
    col_p = jnp.concatenate([col, pad]).reshape(EP // CHUNK, CHUNK)
    x_p = jnp.pad(x, ((0, NP - N), (0, 0)))
    zeros128 = jnp.zeros((NP, D), jnp.float32)
    ones128 = jnp.ones((CHUNK, D), jnp.float32)
    b1r = b1.reshape(1, D)
    b2r = b2.reshape(1, D)

    hist = _hist_kernel(col_p, ones128, zeros128)
    h1, hs1, dis16 = _mm_scale(x_p, W1, hist[0], hist[1])
    acc1 = _agg_kernel(hs1, row_p, col_p, zeros128)
    h2, hs2 = _combine_mm(acc1[0], acc1[1], dis16, h1, b1r, W2)
    acc2 = _agg_kernel(hs2, row_p, col_p, zeros128)
    out = _final(acc2[0], acc2[1], dis16, h2, b2r)
    return out[:N]
